# baseline (speedup 1.0000x reference)
import functools

import jax
import jax.numpy as jnp
from jax import lax
from jax.experimental import pallas as pl
from jax.experimental.pallas import tpu as pltpu
from jax.experimental.pallas import tpu_sc as plsc

NC = 2
NS = 16
CHUNK = 128


def _mesh():
  return plsc.VectorSubcoreMesh(core_axis_name="c", subcore_axis_name="s")


def _make_deg_kernel(NP, EP):
  ch_per_tile = EP // (NC * NS * CHUNK)
  rows_per_tile = NP // NS

  @functools.partial(
      pl.kernel,
      out_type=jax.ShapeDtypeStruct((NC, NP, 128), jnp.float32),
      mesh=_mesh(),
      scratch_types=[
          pltpu.VMEM_SHARED((NP, 128), jnp.float32),
          pltpu.VMEM((CHUNK,), jnp.int32),
          pltpu.VMEM((CHUNK, 128), jnp.float32),
      ],
  )
  def deg_kernel(dst_hbm, ones_hbm, zeros_hbm, out_hbm, acc_sh, dst_v,
                 ones_v):
    c = lax.axis_index("c")
    s = lax.axis_index("s")
    pltpu.sync_copy(zeros_hbm,
                    acc_sh.at[pl.ds(s * rows_per_tile, rows_per_tile)])
    pltpu.sync_copy(ones_hbm, ones_v)
    plsc.subcore_barrier()

    wid = c * NS + s
    base0 = wid * (ch_per_tile * CHUNK)

    def body(j, carry):
      base = base0 + j * CHUNK
      pltpu.sync_copy(dst_hbm.at[pl.ds(base, CHUNK)], dst_v)
      pltpu.sync_copy(ones_v, acc_sh.at[dst_v], add=True)
      return carry

    lax.fori_loop(0, ch_per_tile, body, 0)
    plsc.subcore_barrier()
    pltpu.sync_copy(acc_sh.at[pl.ds(s * rows_per_tile, rows_per_tile)],
                    out_hbm.at[c].at[pl.ds(s * rows_per_tile, rows_per_tile)])

  return deg_kernel


def _make_agg_kernel(N, NP, EP, D):
  ch_per_tile = EP // (NC * NS * CHUNK)
  rows_per_tile = NP // NS

  @functools.partial(
      pl.kernel,
      out_type=jax.ShapeDtypeStruct((NC, NP, D), jnp.float32),
      mesh=_mesh(),
      scratch_types=[
          pltpu.VMEM_SHARED((NP, D), jnp.float32),
          pltpu.VMEM((CHUNK,), jnp.int32),
          pltpu.VMEM((CHUNK,), jnp.int32),
          pltpu.VMEM((CHUNK,), jnp.int32),
          pltpu.VMEM((CHUNK,), jnp.int32),
          pltpu.VMEM((CHUNK, D), jnp.float32),
          pltpu.VMEM((CHUNK, D), jnp.float32),
          pltpu.SemaphoreType.DMA,
          pltpu.SemaphoreType.DMA,
      ],
  )
  def agg_kernel(u_hbm, src_hbm, dst_hbm, zeros_hbm, out_hbm, acc_sh, src0,
                 src1, dst0, dst1, rows0, rows1, sem0, sem1):
    c = lax.axis_index("c")
    s = lax.axis_index("s")
    pltpu.sync_copy(zeros_hbm,
                    acc_sh.at[pl.ds(s * rows_per_tile, rows_per_tile)])
    plsc.subcore_barrier()

    wid = c * NS + s
    base0 = wid * (ch_per_tile * CHUNK)
    srcs = (src0, src1)
    dsts = (dst0, dst1)
    rows = (rows0, rows1)
    sems = (sem0, sem1)
    G = ch_per_tile

    pltpu.sync_copy(src_hbm.at[pl.ds(base0, CHUNK)], src0)
    pltpu.sync_copy(dst_hbm.at[pl.ds(base0, CHUNK)], dst0)
    pltpu.make_async_copy(u_hbm.at[src0], rows0, sem0).start()

    def body(g, carry):
      for b in (0, 1):
        j = 2 * g + b
        nb = 1 - b
        nxt = j + 1 < G

        @pl.when(nxt)
        def _stage():
          base = base0 + (j + 1) * CHUNK
          pltpu.sync_copy(src_hbm.at[pl.ds(base, CHUNK)], srcs[nb])
          pltpu.sync_copy(dst_hbm.at[pl.ds(base, CHUNK)], dsts[nb])

        pltpu.make_async_copy(u_hbm.at[srcs[b]], rows[b], sems[b]).wait()

        @pl.when(nxt)
        def _fire():
          pltpu.make_async_copy(u_hbm.at[srcs[nb]], rows[nb], sems[nb]).start()

        pltpu.sync_copy(rows[b], acc_sh.at[dsts[b]], add=True)
      return carry

    lax.fori_loop(0, G // 2, body, 0)
    plsc.subcore_barrier()
    pltpu.sync_copy(acc_sh.at[pl.ds(s * rows_per_tile, rows_per_tile)],
                    out_hbm.at[c].at[pl.ds(s * rows_per_tile, rows_per_tile)])

  return agg_kernel


def _tc_mm(x, W1):
  def body(x_ref, w_ref, h_ref):
    h_ref[...] = jnp.dot(x_ref[...], w_ref[...],
                         preferred_element_type=jnp.float32)

  return pl.pallas_call(
      body,
      out_shape=jax.ShapeDtypeStruct((x.shape[0], W1.shape[1]), jnp.float32),
  )(x, W1)


def _tc_scale(deg2, h, N):
  def body(d_ref, h_ref, dinv_ref, u_ref):
    deg = d_ref[0, 0:N, 0:1] + d_ref[1, 0:N, 0:1] + 1.0
    dinv = lax.rsqrt(deg)
    dinv_ref[...] = dinv
    u_ref[...] = h_ref[...] * dinv

  return pl.pallas_call(
      body,
      out_shape=(
          jax.ShapeDtypeStruct((N, 1), jnp.float32),
          jax.ShapeDtypeStruct((N, h.shape[1]), jnp.float32),
      ),
  )(deg2, h)


def _tc_layer(s2, u, dinv, b, g, be, Wn, N, D):
  def body(s_ref, u_ref, dinv_ref, b_ref, g_ref, be_ref, w_ref, out_ref):
    conv = (s_ref[0, 0:N, :] + s_ref[1, 0:N, :] + u_ref[...]) * dinv_ref[...]
    conv = conv + b_ref[...]
    mu = jnp.mean(conv, axis=0, keepdims=True)
    d = conv - mu
    var = jnp.mean(d * d, axis=0, keepdims=True)
    z = g_ref[...] * d * lax.rsqrt(var + 1e-5) + be_ref[...]
    z = jnp.maximum(z, 0.0)
    out_ref[...] = (
        jnp.dot(z, w_ref[...], preferred_element_type=jnp.float32)
        * dinv_ref[...])

  return pl.pallas_call(
      body,
      out_shape=jax.ShapeDtypeStruct((N, Wn.shape[1]), jnp.float32),
  )(s2, u, dinv, b.reshape(1, -1), g.reshape(1, -1), be.reshape(1, -1), Wn)


def _tc_final(s2, u, dinv, b, g, be, Wout, bout, N):
  def body(s_ref, u_ref, dinv_ref, b_ref, g_ref, be_ref, w_ref, bo_ref,
           out_ref):
    conv = (s_ref[0, 0:N, :] + s_ref[1, 0:N, :] + u_ref[...]) * dinv_ref[...]
    conv = conv + b_ref[...]
    mu = jnp.mean(conv, axis=0, keepdims=True)
    d = conv - mu
    var = jnp.mean(d * d, axis=0, keepdims=True)
    z = g_ref[...] * d * lax.rsqrt(var + 1e-5) + be_ref[...]
    z = jnp.maximum(z, 0.0)
    out_ref[...] = (
        jnp.dot(z, w_ref[...], preferred_element_type=jnp.float32)
        + bo_ref[...])

  return pl.pallas_call(
      body,
      out_shape=jax.ShapeDtypeStruct((N, Wout.shape[1]), jnp.float32),
  )(s2, u, dinv, b.reshape(1, -1), g.reshape(1, -1), be.reshape(1, -1), Wout,
    bout.reshape(1, -1))


def kernel(x, edge_index, W1, b1, g1, be1, W2, b2, g2, be2, W3, b3, g3, be3,
           Wout, bout):
  N, D = x.shape
  E = edge_index.shape[1]
  NP = ((N + 1 + NS * 8 - 1) // (NS * 8)) * (NS * 8)
  per_round = NC * NS * CHUNK * 2
  EP = ((E + per_round - 1) // per_round) * per_round

  src = edge_index[0]
  dst = edge_index[1]
  pad = EP - E
  srcp = jnp.concatenate([src, jnp.zeros((pad,), jnp.int32)])
  dstp = jnp.concatenate([dst, jnp.full((pad,), N, jnp.int32)])

  rows_per_tile = NP // NS
  ones128 = jnp.ones((CHUNK, 128), jnp.float32)
  zerosD = jnp.zeros((rows_per_tile, D), jnp.float32)

  deg_kernel = _make_deg_kernel(NP, EP)
  agg_kernel = _make_agg_kernel(N, NP, EP, D)

  deg2 = deg_kernel(dstp, ones128, zerosD)
  h1 = _tc_mm(x, W1)
  dinv, u1 = _tc_scale(deg2, h1, N)

  s1 = agg_kernel(u1, srcp, dstp, zerosD)
  u2 = _tc_layer(s1, u1, dinv, b1, g1, be1, W2, N, D)

  s2 = agg_kernel(u2, srcp, dstp, zerosD)
  u3 = _tc_layer(s2, u2, dinv, b2, g2, be2, W3, N, D)

  s3 = agg_kernel(u3, srcp, dstp, zerosD)
  out = _tc_final(s3, u3, dinv, b3, g3, be3, Wout, bout, N)
  return out


# speedup vs baseline: 1.0293x; 1.0293x over previous
import functools

import jax
import jax.numpy as jnp
from jax import lax
from jax.experimental import pallas as pl
from jax.experimental.pallas import tpu as pltpu
from jax.experimental.pallas import tpu_sc as plsc

NC = 2
NS = 16
CHUNK = 128


def _mesh():
  return plsc.VectorSubcoreMesh(core_axis_name="c", subcore_axis_name="s")


def _make_deg_kernel(NP, EP):
  ch_per_tile = EP // (NC * NS * CHUNK)
  rows_per_tile = NP // NS

  @functools.partial(
      pl.kernel,
      out_type=jax.ShapeDtypeStruct((NC, NP, 128), jnp.float32),
      mesh=_mesh(),
      scratch_types=[
          pltpu.VMEM_SHARED((NP, 128), jnp.float32),
          pltpu.VMEM((CHUNK,), jnp.int32),
          pltpu.VMEM((CHUNK, 128), jnp.float32),
      ],
  )
  def deg_kernel(dst_hbm, ones_hbm, zeros_hbm, out_hbm, acc_sh, dst_v,
                 ones_v):
    c = lax.axis_index("c")
    s = lax.axis_index("s")
    pltpu.sync_copy(zeros_hbm,
                    acc_sh.at[pl.ds(s * rows_per_tile, rows_per_tile)])
    pltpu.sync_copy(ones_hbm, ones_v)
    plsc.subcore_barrier()

    wid = c * NS + s
    base0 = wid * (ch_per_tile * CHUNK)

    def body(j, carry):
      base = base0 + j * CHUNK
      pltpu.sync_copy(dst_hbm.at[pl.ds(base, CHUNK)], dst_v)
      pltpu.sync_copy(ones_v, acc_sh.at[dst_v], add=True)
      return carry

    lax.fori_loop(0, ch_per_tile, body, 0)
    plsc.subcore_barrier()
    pltpu.sync_copy(acc_sh.at[pl.ds(s * rows_per_tile, rows_per_tile)],
                    out_hbm.at[c].at[pl.ds(s * rows_per_tile, rows_per_tile)])

  return deg_kernel


def _make_agg_kernel(N, NP, EP, D):
  ch_per_tile = EP // (NC * NS * CHUNK)
  rows_per_tile = NP // NS

  @functools.partial(
      pl.kernel,
      out_type=jax.ShapeDtypeStruct((NC, NP, D), jnp.float32),
      mesh=_mesh(),
      scratch_types=[
          pltpu.VMEM_SHARED((NP, D), jnp.float32),
          pltpu.VMEM((CHUNK,), jnp.int32),
          pltpu.VMEM((CHUNK,), jnp.int32),
          pltpu.VMEM((CHUNK,), jnp.int32),
          pltpu.VMEM((CHUNK,), jnp.int32),
          pltpu.VMEM((CHUNK, D), jnp.float32),
          pltpu.VMEM((CHUNK, D), jnp.float32),
          pltpu.SemaphoreType.DMA,
          pltpu.SemaphoreType.DMA,
      ],
  )
  def agg_kernel(u_hbm, src_hbm, dst_hbm, zeros_hbm, out_hbm, acc_sh, src0,
                 src1, dst0, dst1, rows0, rows1, sem0, sem1):
    c = lax.axis_index("c")
    s = lax.axis_index("s")
    pltpu.sync_copy(zeros_hbm,
                    acc_sh.at[pl.ds(s * rows_per_tile, rows_per_tile)])
    plsc.subcore_barrier()

    wid = c * NS + s
    base0 = wid * (ch_per_tile * CHUNK)
    srcs = (src0, src1)
    dsts = (dst0, dst1)
    rows = (rows0, rows1)
    sems = (sem0, sem1)
    G = ch_per_tile

    pltpu.sync_copy(src_hbm.at[pl.ds(base0, CHUNK)], src0)
    pltpu.sync_copy(dst_hbm.at[pl.ds(base0, CHUNK)], dst0)
    pltpu.make_async_copy(u_hbm.at[src0], rows0, sem0).start()

    def body(g, carry):
      for b in (0, 1):
        j = 2 * g + b
        nb = 1 - b
        nxt = j + 1 < G

        @pl.when(nxt)
        def _stage():
          base = base0 + (j + 1) * CHUNK
          pltpu.sync_copy(src_hbm.at[pl.ds(base, CHUNK)], srcs[nb])
          pltpu.sync_copy(dst_hbm.at[pl.ds(base, CHUNK)], dsts[nb])

        pltpu.make_async_copy(u_hbm.at[srcs[b]], rows[b], sems[b]).wait()

        @pl.when(nxt)
        def _fire():
          pltpu.make_async_copy(u_hbm.at[srcs[nb]], rows[nb], sems[nb]).start()

        pltpu.sync_copy(rows[b], acc_sh.at[dsts[b]], add=True)
      return carry

    lax.fori_loop(0, G // 2, body, 0)
    plsc.subcore_barrier()
    pltpu.sync_copy(acc_sh.at[pl.ds(s * rows_per_tile, rows_per_tile)],
                    out_hbm.at[c].at[pl.ds(s * rows_per_tile, rows_per_tile)])

  return agg_kernel


def _tc_pre(deg2, x, W1, N):
  def body(d_ref, x_ref, w_ref, dinv_ref, u_ref):
    deg = d_ref[0, 0:N, 0:1] + d_ref[1, 0:N, 0:1] + 1.0
    dinv = lax.rsqrt(deg)
    dinv_ref[...] = dinv
    h = jnp.dot(x_ref[...], w_ref[...], preferred_element_type=jnp.float32)
    u_ref[...] = h * dinv

  return pl.pallas_call(
      body,
      out_shape=(
          jax.ShapeDtypeStruct((N, 1), jnp.float32),
          jax.ShapeDtypeStruct((N, x.shape[1]), jnp.float32),
      ),
  )(deg2, x, W1)


def _tc_layer(s2, u, dinv, b, g, be, Wn, N, D):
  def body(s_ref, u_ref, dinv_ref, b_ref, g_ref, be_ref, w_ref, out_ref):
    conv = (s_ref[0, 0:N, :] + s_ref[1, 0:N, :] + u_ref[...]) * dinv_ref[...]
    conv = conv + b_ref[...]
    mu = jnp.mean(conv, axis=0, keepdims=True)
    d = conv - mu
    var = jnp.mean(d * d, axis=0, keepdims=True)
    z = g_ref[...] * d * lax.rsqrt(var + 1e-5) + be_ref[...]
    z = jnp.maximum(z, 0.0)
    out_ref[...] = (
        jnp.dot(z, w_ref[...], preferred_element_type=jnp.float32)
        * dinv_ref[...])

  return pl.pallas_call(
      body,
      out_shape=jax.ShapeDtypeStruct((N, Wn.shape[1]), jnp.float32),
  )(s2, u, dinv, b.reshape(1, -1), g.reshape(1, -1), be.reshape(1, -1), Wn)


def _tc_final(s2, u, dinv, b, g, be, Wout, bout, N):
  def body(s_ref, u_ref, dinv_ref, b_ref, g_ref, be_ref, w_ref, bo_ref,
           out_ref):
    conv = (s_ref[0, 0:N, :] + s_ref[1, 0:N, :] + u_ref[...]) * dinv_ref[...]
    conv = conv + b_ref[...]
    mu = jnp.mean(conv, axis=0, keepdims=True)
    d = conv - mu
    var = jnp.mean(d * d, axis=0, keepdims=True)
    z = g_ref[...] * d * lax.rsqrt(var + 1e-5) + be_ref[...]
    z = jnp.maximum(z, 0.0)
    out_ref[...] = (
        jnp.dot(z, w_ref[...], preferred_element_type=jnp.float32)
        + bo_ref[...])

  return pl.pallas_call(
      body,
      out_shape=jax.ShapeDtypeStruct((N, Wout.shape[1]), jnp.float32),
  )(s2, u, dinv, b.reshape(1, -1), g.reshape(1, -1), be.reshape(1, -1), Wout,
    bout.reshape(1, -1))


def kernel(x, edge_index, W1, b1, g1, be1, W2, b2, g2, be2, W3, b3, g3, be3,
           Wout, bout):
  N, D = x.shape
  E = edge_index.shape[1]
  NP = ((N + 1 + NS * 8 - 1) // (NS * 8)) * (NS * 8)
  per_round = NC * NS * CHUNK * 2
  EP = ((E + per_round - 1) // per_round) * per_round

  src = edge_index[0]
  dst = edge_index[1]
  pad = EP - E
  srcp = jnp.concatenate([src, jnp.zeros((pad,), jnp.int32)])
  dstp = jnp.concatenate([dst, jnp.full((pad,), N, jnp.int32)])

  rows_per_tile = NP // NS
  ones128 = jnp.ones((CHUNK, 128), jnp.float32)
  zerosD = jnp.zeros((rows_per_tile, D), jnp.float32)

  deg_kernel = _make_deg_kernel(NP, EP)
  agg_kernel = _make_agg_kernel(N, NP, EP, D)

  deg2 = deg_kernel(dstp, ones128, zerosD)
  dinv, u1 = _tc_pre(deg2, x, W1, N)

  s1 = agg_kernel(u1, srcp, dstp, zerosD)
  u2 = _tc_layer(s1, u1, dinv, b1, g1, be1, W2, N, D)

  s2 = agg_kernel(u2, srcp, dstp, zerosD)
  u3 = _tc_layer(s2, u2, dinv, b2, g2, be2, W3, N, D)

  s3 = agg_kernel(u3, srcp, dstp, zerosD)
  out = _tc_final(s3, u3, dinv, b3, g3, be3, Wout, bout, N)
  return out


# speedup vs baseline: 1.2513x; 1.2156x over previous
import functools

import jax
import jax.numpy as jnp
from jax import lax
from jax.experimental import pallas as pl
from jax.experimental.pallas import tpu as pltpu
from jax.experimental.pallas import tpu_sc as plsc

NC = 2
NS = 16
CHUNK = 128


def _mesh():
  return plsc.VectorSubcoreMesh(core_axis_name="c", subcore_axis_name="s")


def _make_deg_kernel(NP, EP):
  ch_per_tile = EP // (NC * NS * CHUNK)
  rows_per_tile = NP // NS

  @functools.partial(
      pl.kernel,
      out_type=jax.ShapeDtypeStruct((NC, NP, 128), jnp.float32),
      mesh=_mesh(),
      scratch_types=[
          pltpu.VMEM_SHARED((NP, 128), jnp.float32),
          pltpu.VMEM((CHUNK,), jnp.int32),
          pltpu.VMEM((CHUNK, 128), jnp.float32),
      ],
  )
  def deg_kernel(dst_hbm, ones_hbm, zeros_hbm, out_hbm, acc_sh, dst_v,
                 ones_v):
    c = lax.axis_index("c")
    s = lax.axis_index("s")
    pltpu.sync_copy(zeros_hbm,
                    acc_sh.at[pl.ds(s * rows_per_tile, rows_per_tile)])
    pltpu.sync_copy(ones_hbm, ones_v)
    plsc.subcore_barrier()

    wid = c * NS + s
    base0 = wid * (ch_per_tile * CHUNK)

    def body(j, carry):
      base = base0 + j * CHUNK
      pltpu.sync_copy(dst_hbm.at[pl.ds(base, CHUNK)], dst_v)
      pltpu.sync_copy(ones_v, acc_sh.at[dst_v], add=True)
      return carry

    lax.fori_loop(0, ch_per_tile, body, 0)
    plsc.subcore_barrier()
    pltpu.sync_copy(acc_sh.at[pl.ds(s * rows_per_tile, rows_per_tile)],
                    out_hbm.at[c].at[pl.ds(s * rows_per_tile, rows_per_tile)])

  return deg_kernel


def _make_agg_kernel(N, NP, EP, D, ch0, ch1):
  rows_per_tile = NP // NS

  @functools.partial(
      pl.kernel,
      out_type=jax.ShapeDtypeStruct((NC, NP, D), jnp.float32),
      mesh=_mesh(),
      scratch_types=[
          pltpu.VMEM_SHARED((NP, D), jnp.float32),
          pltpu.VMEM((CHUNK,), jnp.int32),
          pltpu.VMEM((CHUNK,), jnp.int32),
          pltpu.VMEM((CHUNK,), jnp.int32),
          pltpu.VMEM((CHUNK,), jnp.int32),
          pltpu.VMEM((CHUNK, D), jnp.float32),
          pltpu.VMEM((CHUNK, D), jnp.float32),
          pltpu.SemaphoreType.DMA,
          pltpu.SemaphoreType.DMA,
      ],
  )
  def agg_kernel(u_hbm, src_hbm, dst_hbm, zeros_hbm, out_hbm, acc_sh, src0,
                 src1, dst0, dst1, rows0, rows1, sem0, sem1):
    c = lax.axis_index("c")
    s = lax.axis_index("s")
    pltpu.sync_copy(zeros_hbm,
                    acc_sh.at[pl.ds(s * rows_per_tile, rows_per_tile)])
    plsc.subcore_barrier()

    base0 = jnp.where(c == 0, s * ch0, NS * ch0 + s * ch1) * CHUNK
    srcs = (src0, src1)
    dsts = (dst0, dst1)
    rows = (rows0, rows1)
    sems = (sem0, sem1)
    G = jnp.where(c == 0, ch0, ch1)

    pltpu.sync_copy(src_hbm.at[pl.ds(base0, CHUNK)], src0)
    pltpu.sync_copy(dst_hbm.at[pl.ds(base0, CHUNK)], dst0)
    pltpu.make_async_copy(u_hbm.at[src0], rows0, sem0).start()

    def body(g, carry):
      for b in (0, 1):
        j = 2 * g + b
        nb = 1 - b
        nxt = j + 1 < G

        @pl.when(nxt)
        def _stage():
          base = base0 + (j + 1) * CHUNK
          pltpu.sync_copy(src_hbm.at[pl.ds(base, CHUNK)], srcs[nb])
          pltpu.sync_copy(dst_hbm.at[pl.ds(base, CHUNK)], dsts[nb])

        pltpu.make_async_copy(u_hbm.at[srcs[b]], rows[b], sems[b]).wait()

        @pl.when(nxt)
        def _fire():
          pltpu.make_async_copy(u_hbm.at[srcs[nb]], rows[nb], sems[nb]).start()

        pltpu.sync_copy(rows[b], acc_sh.at[dsts[b]], add=True)
      return carry

    lax.fori_loop(0, G // 2, body, 0)
    plsc.subcore_barrier()
    pltpu.sync_copy(acc_sh.at[pl.ds(s * rows_per_tile, rows_per_tile)],
                    out_hbm.at[c].at[pl.ds(s * rows_per_tile, rows_per_tile)])

  return agg_kernel


def _tc_pre(deg2, x, W1, N):
  def body(d_ref, x_ref, w_ref, dinv_ref, u_ref):
    deg = d_ref[0, 0:N, 0:1] + d_ref[1, 0:N, 0:1] + 1.0
    dinv = lax.rsqrt(deg)
    dinv_ref[...] = dinv
    h = jnp.dot(x_ref[...], w_ref[...], preferred_element_type=jnp.float32)
    u_ref[...] = h * dinv

  return pl.pallas_call(
      body,
      out_shape=(
          jax.ShapeDtypeStruct((N, 1), jnp.float32),
          jax.ShapeDtypeStruct((N, x.shape[1]), jnp.float32),
      ),
  )(deg2, x, W1)


def _tc_layer(s2, u, dinv, b, g, be, Wn, N, D):
  def body(s_ref, u_ref, dinv_ref, b_ref, g_ref, be_ref, w_ref, out_ref):
    conv = (s_ref[0, 0:N, :] + s_ref[1, 0:N, :] + u_ref[...]) * dinv_ref[...]
    conv = conv + b_ref[...]
    mu = jnp.mean(conv, axis=0, keepdims=True)
    d = conv - mu
    var = jnp.mean(d * d, axis=0, keepdims=True)
    z = g_ref[...] * d * lax.rsqrt(var + 1e-5) + be_ref[...]
    z = jnp.maximum(z, 0.0)
    out_ref[...] = (
        jnp.dot(z, w_ref[...], preferred_element_type=jnp.float32)
        * dinv_ref[...])

  return pl.pallas_call(
      body,
      out_shape=jax.ShapeDtypeStruct((N, Wn.shape[1]), jnp.float32),
  )(s2, u, dinv, b.reshape(1, -1), g.reshape(1, -1), be.reshape(1, -1), Wn)


def _tc_final(s2, u, dinv, b, g, be, Wout, bout, N):
  def body(s_ref, u_ref, dinv_ref, b_ref, g_ref, be_ref, w_ref, bo_ref,
           out_ref):
    conv = (s_ref[0, 0:N, :] + s_ref[1, 0:N, :] + u_ref[...]) * dinv_ref[...]
    conv = conv + b_ref[...]
    mu = jnp.mean(conv, axis=0, keepdims=True)
    d = conv - mu
    var = jnp.mean(d * d, axis=0, keepdims=True)
    z = g_ref[...] * d * lax.rsqrt(var + 1e-5) + be_ref[...]
    z = jnp.maximum(z, 0.0)
    out_ref[...] = (
        jnp.dot(z, w_ref[...], preferred_element_type=jnp.float32)
        + bo_ref[...])

  return pl.pallas_call(
      body,
      out_shape=jax.ShapeDtypeStruct((N, Wout.shape[1]), jnp.float32),
  )(s2, u, dinv, b.reshape(1, -1), g.reshape(1, -1), be.reshape(1, -1), Wout,
    bout.reshape(1, -1))


def kernel(x, edge_index, W1, b1, g1, be1, W2, b2, g2, be2, W3, b3, g3, be3,
           Wout, bout):
  N, D = x.shape
  E = edge_index.shape[1]
  NP = ((N + 1 + NS * 8 - 1) // (NS * 8)) * (NS * 8)
  chunks = (E + CHUNK - 1) // CHUNK
  chpp = (chunks + NS - 1) // NS
  ch0 = max(2, 2 * int(chpp * 0.37 / 2 + 0.5))
  ch1 = max(2, 2 * ((chpp - ch0 + 1) // 2))
  EP = NS * (ch0 + ch1) * CHUNK

  src = edge_index[0]
  dst = edge_index[1]
  pad = EP - E
  srcp = jnp.concatenate([src, jnp.zeros((pad,), jnp.int32)])
  dstp = jnp.concatenate([dst, jnp.full((pad,), N, jnp.int32)])

  rows_per_tile = NP // NS
  ones128 = jnp.ones((CHUNK, 128), jnp.float32)
  zerosD = jnp.zeros((rows_per_tile, D), jnp.float32)

  deg_kernel = _make_deg_kernel(NP, EP)
  agg_kernel = _make_agg_kernel(N, NP, EP, D, ch0, ch1)

  deg2 = deg_kernel(dstp, ones128, zerosD)
  dinv, u1 = _tc_pre(deg2, x, W1, N)

  s1 = agg_kernel(u1, srcp, dstp, zerosD)
  u2 = _tc_layer(s1, u1, dinv, b1, g1, be1, W2, N, D)

  s2 = agg_kernel(u2, srcp, dstp, zerosD)
  u3 = _tc_layer(s2, u2, dinv, b2, g2, be2, W3, N, D)

  s3 = agg_kernel(u3, srcp, dstp, zerosD)
  out = _tc_final(s3, u3, dinv, b3, g3, be3, Wout, bout, N)
  return out


# speedup vs baseline: 1.3023x; 1.0408x over previous
import functools

import jax
import jax.numpy as jnp
from jax import lax
from jax.experimental import pallas as pl
from jax.experimental.pallas import tpu as pltpu
from jax.experimental.pallas import tpu_sc as plsc

NC = 2
NS = 16
CHUNK = 128


def _mesh():
  return plsc.VectorSubcoreMesh(core_axis_name="c", subcore_axis_name="s")


def _make_deg_kernel(NP, EP):
  ch_per_tile = EP // (NC * NS * CHUNK)
  rows_per_tile = NP // NS

  @functools.partial(
      pl.kernel,
      out_type=jax.ShapeDtypeStruct((NC, NP, 128), jnp.float32),
      mesh=_mesh(),
      scratch_types=[
          pltpu.VMEM_SHARED((NP, 128), jnp.float32),
          pltpu.VMEM((CHUNK,), jnp.int32),
          pltpu.VMEM((CHUNK, 128), jnp.float32),
      ],
  )
  def deg_kernel(dst_hbm, ones_hbm, zeros_hbm, out_hbm, acc_sh, dst_v,
                 ones_v):
    c = lax.axis_index("c")
    s = lax.axis_index("s")
    pltpu.sync_copy(zeros_hbm,
                    acc_sh.at[pl.ds(s * rows_per_tile, rows_per_tile)])
    pltpu.sync_copy(ones_hbm, ones_v)
    plsc.subcore_barrier()

    wid = c * NS + s
    base0 = wid * (ch_per_tile * CHUNK)

    def body(j, carry):
      base = base0 + j * CHUNK
      pltpu.sync_copy(dst_hbm.at[pl.ds(base, CHUNK)], dst_v)
      pltpu.sync_copy(ones_v, acc_sh.at[dst_v], add=True)
      return carry

    lax.fori_loop(0, ch_per_tile, body, 0)
    plsc.subcore_barrier()
    pltpu.sync_copy(acc_sh.at[pl.ds(s * rows_per_tile, rows_per_tile)],
                    out_hbm.at[c].at[pl.ds(s * rows_per_tile, rows_per_tile)])

  return deg_kernel


def _make_agg_kernel(N, NP, EP, D, ch0, ch1):
  rows_per_tile = NP // NS

  @functools.partial(
      pl.kernel,
      out_type=jax.ShapeDtypeStruct((NC, NP, D), jnp.float32),
      mesh=_mesh(),
      scratch_types=[
          pltpu.VMEM_SHARED((NP, D), jnp.float32),
          pltpu.VMEM((CHUNK,), jnp.int32),
          pltpu.VMEM((CHUNK,), jnp.int32),
          pltpu.VMEM((CHUNK,), jnp.int32),
          pltpu.VMEM((CHUNK,), jnp.int32),
          pltpu.VMEM((CHUNK, D), jnp.float32),
          pltpu.VMEM((CHUNK, D), jnp.float32),
          pltpu.SemaphoreType.DMA,
          pltpu.SemaphoreType.DMA,
      ],
  )
  def agg_kernel(u_hbm, src_hbm, dst_hbm, zeros_hbm, out_hbm, acc_sh, src0,
                 src1, dst0, dst1, rows0, rows1, sem0, sem1):
    c = lax.axis_index("c")
    s = lax.axis_index("s")
    pltpu.sync_copy(zeros_hbm,
                    acc_sh.at[pl.ds(s * rows_per_tile, rows_per_tile)])
    plsc.subcore_barrier()

    base0 = jnp.where(c == 0, s * ch0, NS * ch0 + s * ch1) * CHUNK
    srcs = (src0, src1)
    dsts = (dst0, dst1)
    rows = (rows0, rows1)
    sems = (sem0, sem1)
    G = jnp.where(c == 0, ch0, ch1)

    pltpu.sync_copy(src_hbm.at[pl.ds(base0, CHUNK)], src0)
    pltpu.sync_copy(dst_hbm.at[pl.ds(base0, CHUNK)], dst0)
    pltpu.make_async_copy(u_hbm.at[src0], rows0, sem0).start()

    def body(g, carry):
      for b in (0, 1):
        j = 2 * g + b
        nb = 1 - b
        nxt = j + 1 < G

        @pl.when(nxt)
        def _stage():
          base = base0 + (j + 1) * CHUNK
          pltpu.sync_copy(src_hbm.at[pl.ds(base, CHUNK)], srcs[nb])
          pltpu.sync_copy(dst_hbm.at[pl.ds(base, CHUNK)], dsts[nb])

        pltpu.make_async_copy(u_hbm.at[srcs[b]], rows[b], sems[b]).wait()

        @pl.when(nxt)
        def _fire():
          pltpu.make_async_copy(u_hbm.at[srcs[nb]], rows[nb], sems[nb]).start()

        pltpu.sync_copy(rows[b], acc_sh.at[dsts[b]], add=True)
      return carry

    lax.fori_loop(0, G // 2, body, 0)
    plsc.subcore_barrier()
    pltpu.sync_copy(acc_sh.at[pl.ds(s * rows_per_tile, rows_per_tile)],
                    out_hbm.at[c].at[pl.ds(s * rows_per_tile, rows_per_tile)])

  return agg_kernel


def _tc_pre(deg2, x, W1, N):
  def body(d_ref, x_ref, w_ref, dinv_ref, u_ref):
    deg = d_ref[0, 0:N, 0:1] + d_ref[1, 0:N, 0:1] + 1.0
    dinv = lax.rsqrt(deg)
    dinv_ref[...] = dinv
    h = jnp.dot(x_ref[...], w_ref[...], preferred_element_type=jnp.float32)
    u_ref[...] = h * dinv

  return pl.pallas_call(
      body,
      out_shape=(
          jax.ShapeDtypeStruct((N, 1), jnp.float32),
          jax.ShapeDtypeStruct((N, x.shape[1]), jnp.float32),
      ),
  )(deg2, x, W1)


def _tc_layer(s2, u, dinv, b, g, be, Wn, N, D):
  def body(s_ref, u_ref, dinv_ref, b_ref, g_ref, be_ref, w_ref, out_ref):
    conv = (s_ref[0, 0:N, :] + s_ref[1, 0:N, :] + u_ref[...]) * dinv_ref[...]
    conv = conv + b_ref[...]
    mu = jnp.mean(conv, axis=0, keepdims=True)
    d = conv - mu
    var = jnp.mean(d * d, axis=0, keepdims=True)
    z = g_ref[...] * d * lax.rsqrt(var + 1e-5) + be_ref[...]
    z = jnp.maximum(z, 0.0)
    out_ref[...] = (
        jnp.dot(z, w_ref[...], preferred_element_type=jnp.float32)
        * dinv_ref[...])

  return pl.pallas_call(
      body,
      out_shape=jax.ShapeDtypeStruct((N, Wn.shape[1]), jnp.float32),
  )(s2, u, dinv, b.reshape(1, -1), g.reshape(1, -1), be.reshape(1, -1), Wn)


def _tc_final(s2, u, dinv, b, g, be, Wout, bout, N):
  def body(s_ref, u_ref, dinv_ref, b_ref, g_ref, be_ref, w_ref, bo_ref,
           out_ref):
    conv = (s_ref[0, 0:N, :] + s_ref[1, 0:N, :] + u_ref[...]) * dinv_ref[...]
    conv = conv + b_ref[...]
    mu = jnp.mean(conv, axis=0, keepdims=True)
    d = conv - mu
    var = jnp.mean(d * d, axis=0, keepdims=True)
    z = g_ref[...] * d * lax.rsqrt(var + 1e-5) + be_ref[...]
    z = jnp.maximum(z, 0.0)
    out_ref[...] = (
        jnp.dot(z, w_ref[...], preferred_element_type=jnp.float32)
        + bo_ref[...])

  return pl.pallas_call(
      body,
      out_shape=jax.ShapeDtypeStruct((N, Wout.shape[1]), jnp.float32),
  )(s2, u, dinv, b.reshape(1, -1), g.reshape(1, -1), be.reshape(1, -1), Wout,
    bout.reshape(1, -1))


def kernel(x, edge_index, W1, b1, g1, be1, W2, b2, g2, be2, W3, b3, g3, be3,
           Wout, bout):
  N, D = x.shape
  E = edge_index.shape[1]
  NP = ((N + 1 + NS * 8 - 1) // (NS * 8)) * (NS * 8)
  chunks = (E + CHUNK - 1) // CHUNK
  chpp = (chunks + NS - 1) // NS
  ch0 = max(2, 2 * int(chpp * 0.44 / 2 + 0.5))
  ch1 = max(2, 2 * ((chpp - ch0 + 1) // 2))
  EP = NS * (ch0 + ch1) * CHUNK

  src = edge_index[0]
  dst = edge_index[1]
  pad = EP - E
  srcp = jnp.concatenate([src, jnp.zeros((pad,), jnp.int32)])
  dstp = jnp.concatenate([dst, jnp.full((pad,), N, jnp.int32)])

  rows_per_tile = NP // NS
  ones128 = jnp.ones((CHUNK, 128), jnp.float32)
  zerosD = jnp.zeros((rows_per_tile, D), jnp.float32)

  deg_kernel = _make_deg_kernel(NP, EP)
  agg_kernel = _make_agg_kernel(N, NP, EP, D, ch0, ch1)

  deg2 = deg_kernel(dstp, ones128, zerosD)
  dinv, u1 = _tc_pre(deg2, x, W1, N)

  s1 = agg_kernel(u1, srcp, dstp, zerosD)
  u2 = _tc_layer(s1, u1, dinv, b1, g1, be1, W2, N, D)

  s2 = agg_kernel(u2, srcp, dstp, zerosD)
  u3 = _tc_layer(s2, u2, dinv, b2, g2, be2, W3, N, D)

  s3 = agg_kernel(u3, srcp, dstp, zerosD)
  out = _tc_final(s3, u3, dinv, b3, g3, be3, Wout, bout, N)
  return out
